# baseline (speedup 1.0000x reference)
import functools
import jax
import jax.numpy as jnp
from jax import lax
from jax.experimental import pallas as pl
from jax.experimental.pallas import tpu as pltpu
from jax.experimental.pallas import tpu_sc as plsc

N = 10000
N_PAD = 10240
E = 160000
E_PAD = 163840
IN_CH = 256
HID = 512
OUT_CH = 256
EPS = 1e-5

_NC = 2
_NS = 16
_ROWS_PER_SUB = N_PAD // _NS
_EDGES_PER_SUB = E_PAD // _NS
_EB = 128
_NB = _EDGES_PER_SUB // _EB


def _sc_mesh():
    return plsc.VectorSubcoreMesh(core_axis_name="c", subcore_axis_name="s")


def _sc_degree(dst2d, zeros128, ones128):
    @functools.partial(
        pl.kernel,
        out_type=jax.ShapeDtypeStruct((N_PAD, 128), jnp.float32),
        mesh=_sc_mesh(),
        scratch_types=[
            pltpu.VMEM_SHARED((N_PAD, 128), jnp.float32),
            pltpu.VMEM((_EB, 128), jnp.float32),
            pltpu.VMEM((_NB, _EB), jnp.int32),
            pltpu.SemaphoreType.DMA,
        ],
    )
    def k(dst_hbm, z_hbm, ones_hbm, out_hbm, acc, ones_v, didx2, ssem):
        cid = lax.axis_index("c")
        sid = lax.axis_index("s")
        pltpu.sync_copy(ones_hbm, ones_v)
        pltpu.sync_copy(dst_hbm.at[pl.ds(sid * _NB, _NB)], didx2)
        pltpu.sync_copy(z_hbm, acc.at[pl.ds(sid * _ROWS_PER_SUB, _ROWS_PER_SUB)])
        plsc.subcore_barrier()

        @pl.loop(0, _NB, step=8)
        def _(b):
            cps = [pltpu.async_copy(ones_v, acc.at[didx2.at[b + j]], ssem,
                                    add=True) for j in range(8)]
            for cp in cps:
                cp.wait()

        plsc.subcore_barrier()

        @pl.when(cid == 0)
        def _():
            s = pl.ds(sid * _ROWS_PER_SUB, _ROWS_PER_SUB)
            pltpu.sync_copy(acc.at[s], out_hbm.at[s])

    return k(dst2d, zeros128, ones128)


_NSPLIT = 4


def _split_gather(table, sidx, bufs, j, sem):
    w = _EB // _NSPLIT
    for h in range(_NSPLIT):
        pltpu.async_copy(table.at[sidx.at[j, pl.ds(h * w, w)]],
                         bufs.at[j, pl.ds(h * w, w)], sem)


def _sc_segsum(hs_c, src2d, dst2d, zeros128, n_chunks):
    per_core = n_chunks // _NC

    @functools.partial(
        pl.kernel,
        out_type=jax.ShapeDtypeStruct((n_chunks, N_PAD, 128), jnp.float32),
        mesh=_sc_mesh(),
        scratch_types=[
            pltpu.VMEM_SHARED((N_PAD, 128), jnp.float32),
            pltpu.VMEM((2, _EB, 128), jnp.float32),
            pltpu.VMEM((_NB, _EB), jnp.int32),
            pltpu.VMEM((2, _EB), jnp.int32),
            [pltpu.SemaphoreType.DMA] * 2,
            [pltpu.SemaphoreType.DMA] * 2,
            [pltpu.SemaphoreType.DMA] * 2,
        ],
    )
    def k(hs_hbm, src_hbm, dst_hbm, z_hbm, out_hbm, acc, bufs, didx2, sidx,
          gsems, ssems, isems):
        cid = lax.axis_index("c")
        sid = lax.axis_index("s")
        rows = pl.ds(sid * _ROWS_PER_SUB, _ROWS_PER_SUB)
        ib = sid * _NB
        pltpu.sync_copy(dst_hbm.at[pl.ds(ib, _NB)], didx2)
        for ci in range(per_core):
            chunk = cid * per_core + ci
            pltpu.sync_copy(z_hbm, acc.at[rows])
            for j in range(2):
                pltpu.sync_copy(src_hbm.at[ib + j], sidx.at[j])
                _split_gather(hs_hbm.at[chunk], sidx, bufs, j, gsems[j])
            plsc.subcore_barrier()

            @pl.loop(0, _NB, step=2)
            def _(b):
                scs, lds = [], []
                for j in range(2):
                    pltpu.make_async_copy(hs_hbm.at[chunk].at[sidx.at[j]],
                                          bufs.at[j], gsems[j]).wait()
                    lds.append(pltpu.async_copy(src_hbm.at[ib + b + 2 + j],
                                                sidx.at[j], isems[j]))
                    scs.append(pltpu.async_copy(
                        bufs.at[j], acc.at[didx2.at[b + j]], ssems[j],
                        add=True))
                for j in range(2):
                    scs[j].wait()
                    lds[j].wait()

                    @pl.when(b + 2 + j < _NB)
                    def _():
                        _split_gather(hs_hbm.at[chunk], sidx, bufs, j,
                                      gsems[j])

            plsc.subcore_barrier()
            pltpu.sync_copy(acc.at[rows], out_hbm.at[chunk].at[rows])

    return k(hs_c, src2d, dst2d, zeros128)


_RB = 1280
_RG = N_PAD // _RB


def _tc_a_body(x_ref, deg_ref, o_ref):
    dinv = lax.rsqrt(deg_ref[:, 0:1] + 1.0)
    o_ref[0] = x_ref[...] * dinv


def _tc_a(x_p, deg16):
    return pl.pallas_call(
        _tc_a_body,
        grid=(_RG, IN_CH // 128),
        in_specs=[
            pl.BlockSpec((_RB, 128), lambda i, j: (i, j)),
            pl.BlockSpec((_RB, 128), lambda i, j: (i, 0)),
        ],
        out_specs=pl.BlockSpec((1, _RB, 128), lambda i, j: (j, i, 0)),
        out_shape=jax.ShapeDtypeStruct((IN_CH // 128, N_PAD, 128), jnp.float32),
    )(x_p, deg16)


def _tc_b_body(sx_ref, xs_ref, deg_ref, w1_ref, b_ref, g_ref, be_ref, w2_ref,
               o_ref):
    dinv = lax.rsqrt(deg_ref[:, 0:1] + 1.0)
    u = (sx_ref[...] + xs_ref[...]) * dinv[None]
    t = jnp.dot(u[0], w1_ref[:128], preferred_element_type=jnp.float32)
    t = t + jnp.dot(u[1], w1_ref[128:], preferred_element_type=jnp.float32)
    t = t + b_ref[...]
    mu = jnp.mean(t, axis=-1, keepdims=True)
    var = jnp.mean((t - mu) ** 2, axis=-1, keepdims=True)
    tn = (t - mu) * lax.rsqrt(var + EPS) * g_ref[...] + be_ref[...]
    tn = jnp.maximum(tn, 0.0)
    h2 = jnp.dot(tn, w2_ref[...], preferred_element_type=jnp.float32)
    row = lax.broadcasted_iota(jnp.int32, (_RB, 1), 0) + pl.program_id(0) * _RB
    hs2 = h2 * dinv * (row < N).astype(jnp.float32)
    o_ref[0] = hs2[:, :128]
    o_ref[1] = hs2[:, 128:]


def _tc_b(sxc, xsc, deg16, W1, b1, g1, be1, W2):
    ci = IN_CH // 128
    c2 = OUT_CH // 128
    return pl.pallas_call(
        _tc_b_body,
        grid=(_RG,),
        in_specs=[
            pl.BlockSpec((ci, _RB, 128), lambda i: (0, i, 0)),
            pl.BlockSpec((ci, _RB, 128), lambda i: (0, i, 0)),
            pl.BlockSpec((_RB, 128), lambda i: (i, 0)),
            pl.BlockSpec((IN_CH, HID), lambda i: (0, 0)),
            pl.BlockSpec((1, HID), lambda i: (0, 0)),
            pl.BlockSpec((1, HID), lambda i: (0, 0)),
            pl.BlockSpec((1, HID), lambda i: (0, 0)),
            pl.BlockSpec((HID, OUT_CH), lambda i: (0, 0)),
        ],
        out_specs=pl.BlockSpec((c2, _RB, 128), lambda i: (0, i, 0)),
        out_shape=jax.ShapeDtypeStruct((c2, N_PAD, 128), jnp.float32),
    )(sxc, xsc, deg16, W1, b1.reshape(1, HID), g1.reshape(1, HID),
      be1.reshape(1, HID), W2)


def _tc_c_body(agg_ref, hs_ref, deg_ref, b_ref, g_ref, be_ref, o_ref):
    dinv = lax.rsqrt(deg_ref[:, 0:1] + 1.0)
    t = (agg_ref[...] + hs_ref[...]) * dinv[None] + b_ref[...][:, None, :]
    mu = jnp.mean(t, axis=(0, 2))[None, :, None]
    var = jnp.mean((t - mu) ** 2, axis=(0, 2))[None, :, None]
    tn = (t - mu) * lax.rsqrt(var + EPS)
    tn = jnp.maximum(tn * g_ref[...][:, None, :] + be_ref[...][:, None, :], 0.0)
    o_ref[...] = jnp.concatenate([tn[0], tn[1]], axis=1)


def _tc_c(agg2c, hs2c, deg16, b2, g2, be2):
    c2 = OUT_CH // 128
    return pl.pallas_call(
        _tc_c_body,
        grid=(_RG,),
        in_specs=[
            pl.BlockSpec((c2, _RB, 128), lambda i: (0, i, 0)),
            pl.BlockSpec((c2, _RB, 128), lambda i: (0, i, 0)),
            pl.BlockSpec((_RB, 128), lambda i: (i, 0)),
            pl.BlockSpec((c2, 128), lambda i: (0, 0)),
            pl.BlockSpec((c2, 128), lambda i: (0, 0)),
            pl.BlockSpec((c2, 128), lambda i: (0, 0)),
        ],
        out_specs=pl.BlockSpec((_RB, OUT_CH), lambda i: (i, 0)),
        out_shape=jax.ShapeDtypeStruct((N_PAD, OUT_CH), jnp.float32),
    )(agg2c, hs2c, deg16, b2.reshape(c2, 128), g2.reshape(c2, 128),
      be2.reshape(c2, 128))


def kernel(x, edge_index, W1, b1, g1, be1, W2, b2, g2, be2):
    ei = edge_index.astype(jnp.int32)
    pad = jnp.full((E_PAD - E,), N, jnp.int32)
    src2d = jnp.pad(jnp.concatenate([ei[0], pad]).reshape(E_PAD // _EB, _EB),
                    ((0, 2), (0, 0)))
    dst2d = jnp.concatenate([ei[1], pad]).reshape(E_PAD // _EB, _EB)
    x_p = jnp.pad(x, ((0, N_PAD - N), (0, 0)))
    zeros128 = jnp.zeros((_ROWS_PER_SUB, 128), jnp.float32)
    ones128 = jnp.ones((_EB, 128), jnp.float32)

    deg16 = _sc_degree(dst2d, zeros128, ones128)
    xsc = _tc_a(x_p, deg16)
    sxc = _sc_segsum(xsc, src2d, dst2d, zeros128, IN_CH // 128)
    hs2c = _tc_b(sxc, xsc, deg16, W1, b1, g1, be1, W2)
    agg2c = _sc_segsum(hs2c, src2d, dst2d, zeros128, OUT_CH // 128)
    out_p = _tc_c(agg2c, hs2c, deg16, b2, g2, be2)
    return out_p[:N]


# speedup vs baseline: 1.0360x; 1.0360x over previous
import functools
import jax
import jax.numpy as jnp
from jax import lax
from jax.experimental import pallas as pl
from jax.experimental.pallas import tpu as pltpu
from jax.experimental.pallas import tpu_sc as plsc

N = 10000
N_PAD = 10240
E = 160000
E_PAD = 163840
IN_CH = 256
HID = 512
OUT_CH = 256
EPS = 1e-5

_NC = 2
_NS = 16
_ROWS_PER_SUB = N_PAD // _NS
_EDGES_PER_SUB = E_PAD // _NS
_EB = 128
_NB = _EDGES_PER_SUB // _EB


def _sc_mesh():
    return plsc.VectorSubcoreMesh(core_axis_name="c", subcore_axis_name="s")


def _sc_degree(dst2d, zeros128, ones128):
    half = _NB // 2

    @functools.partial(
        pl.kernel,
        out_type=jax.ShapeDtypeStruct((_NC, N_PAD, 128), jnp.float32),
        mesh=_sc_mesh(),
        scratch_types=[
            pltpu.VMEM_SHARED((N_PAD, 128), jnp.float32),
            pltpu.VMEM((_EB, 128), jnp.float32),
            pltpu.VMEM((half, _EB), jnp.int32),
            pltpu.SemaphoreType.DMA,
        ],
    )
    def k(dst_hbm, z_hbm, ones_hbm, out_hbm, acc, ones_v, didx2, ssem):
        cid = lax.axis_index("c")
        sid = lax.axis_index("s")
        pltpu.sync_copy(ones_hbm, ones_v)
        pltpu.sync_copy(dst_hbm.at[pl.ds((cid * _NS + sid) * half, half)],
                        didx2)
        pltpu.sync_copy(z_hbm, acc.at[pl.ds(sid * _ROWS_PER_SUB, _ROWS_PER_SUB)])
        plsc.subcore_barrier()

        @pl.loop(0, half, step=8)
        def _(b):
            cps = [pltpu.async_copy(ones_v, acc.at[didx2.at[b + j]], ssem,
                                    add=True) for j in range(8)]
            for cp in cps:
                cp.wait()

        plsc.subcore_barrier()
        s = pl.ds(sid * _ROWS_PER_SUB, _ROWS_PER_SUB)
        pltpu.sync_copy(acc.at[s], out_hbm.at[cid].at[s])

    return k(dst2d, zeros128, ones128)


_NSPLIT = 4


def _split_gather(table, sidx, bufs, j, sem):
    w = _EB // _NSPLIT
    for h in range(_NSPLIT):
        pltpu.async_copy(table.at[sidx.at[j, pl.ds(h * w, w)]],
                         bufs.at[j, pl.ds(h * w, w)], sem)


def _sc_segsum(hs_c, src2d, dst2d, zeros128, n_chunks):
    per_core = n_chunks // _NC

    @functools.partial(
        pl.kernel,
        out_type=jax.ShapeDtypeStruct((n_chunks, N_PAD, 128), jnp.float32),
        mesh=_sc_mesh(),
        scratch_types=[
            pltpu.VMEM_SHARED((N_PAD, 128), jnp.float32),
            pltpu.VMEM((2, _EB, 128), jnp.float32),
            pltpu.VMEM((_NB, _EB), jnp.int32),
            pltpu.VMEM((2, _EB), jnp.int32),
            [pltpu.SemaphoreType.DMA] * 2,
            [pltpu.SemaphoreType.DMA] * 2,
            [pltpu.SemaphoreType.DMA] * 2,
        ],
    )
    def k(hs_hbm, src_hbm, dst_hbm, z_hbm, out_hbm, acc, bufs, didx2, sidx,
          gsems, ssems, isems):
        cid = lax.axis_index("c")
        sid = lax.axis_index("s")
        rows = pl.ds(sid * _ROWS_PER_SUB, _ROWS_PER_SUB)
        ib = sid * _NB
        pltpu.sync_copy(dst_hbm.at[pl.ds(ib, _NB)], didx2)
        for ci in range(per_core):
            chunk = cid * per_core + ci
            pltpu.sync_copy(z_hbm, acc.at[rows])
            for j in range(2):
                pltpu.sync_copy(src_hbm.at[ib + j], sidx.at[j])
                _split_gather(hs_hbm.at[chunk], sidx, bufs, j, gsems[j])
            plsc.subcore_barrier()

            @pl.loop(0, _NB, step=2)
            def _(b):
                scs, lds = [], []
                for j in range(2):
                    pltpu.make_async_copy(hs_hbm.at[chunk].at[sidx.at[j]],
                                          bufs.at[j], gsems[j]).wait()
                    lds.append(pltpu.async_copy(src_hbm.at[ib + b + 2 + j],
                                                sidx.at[j], isems[j]))
                    scs.append(pltpu.async_copy(
                        bufs.at[j], acc.at[didx2.at[b + j]], ssems[j],
                        add=True))
                for j in range(2):
                    scs[j].wait()
                    lds[j].wait()

                    @pl.when(b + 2 + j < _NB)
                    def _():
                        _split_gather(hs_hbm.at[chunk], sidx, bufs, j,
                                      gsems[j])

            plsc.subcore_barrier()
            pltpu.sync_copy(acc.at[rows], out_hbm.at[chunk].at[rows])

    return k(hs_c, src2d, dst2d, zeros128)


_RB = 1280
_RG = N_PAD // _RB


def _tc_a_body(x_ref, deg_ref, o_ref):
    dinv = lax.rsqrt(deg_ref[0, :, 0:1] + deg_ref[1, :, 0:1] + 1.0)
    o_ref[0] = x_ref[...] * dinv


def _tc_a(x_p, deg16):
    return pl.pallas_call(
        _tc_a_body,
        grid=(_RG, IN_CH // 128),
        in_specs=[
            pl.BlockSpec((_RB, 128), lambda i, j: (i, j)),
            pl.BlockSpec((_NC, _RB, 128), lambda i, j: (0, i, 0)),
        ],
        out_specs=pl.BlockSpec((1, _RB, 128), lambda i, j: (j, i, 0)),
        out_shape=jax.ShapeDtypeStruct((IN_CH // 128, N_PAD, 128), jnp.float32),
    )(x_p, deg16)


def _tc_b_body(sx_ref, xs_ref, deg_ref, w1_ref, b_ref, g_ref, be_ref, w2_ref,
               o_ref):
    dinv = lax.rsqrt(deg_ref[0, :, 0:1] + deg_ref[1, :, 0:1] + 1.0)
    u = (sx_ref[...] + xs_ref[...]) * dinv[None]
    t = jnp.dot(u[0], w1_ref[:128], preferred_element_type=jnp.float32)
    t = t + jnp.dot(u[1], w1_ref[128:], preferred_element_type=jnp.float32)
    t = t + b_ref[...]
    mu = jnp.mean(t, axis=-1, keepdims=True)
    var = jnp.mean((t - mu) ** 2, axis=-1, keepdims=True)
    tn = (t - mu) * lax.rsqrt(var + EPS) * g_ref[...] + be_ref[...]
    tn = jnp.maximum(tn, 0.0)
    h2 = jnp.dot(tn, w2_ref[...], preferred_element_type=jnp.float32)
    row = lax.broadcasted_iota(jnp.int32, (_RB, 1), 0) + pl.program_id(0) * _RB
    hs2 = h2 * dinv * (row < N).astype(jnp.float32)
    o_ref[0] = hs2[:, :128]
    o_ref[1] = hs2[:, 128:]


def _tc_b(sxc, xsc, deg16, W1, b1, g1, be1, W2):
    ci = IN_CH // 128
    c2 = OUT_CH // 128
    return pl.pallas_call(
        _tc_b_body,
        grid=(_RG,),
        in_specs=[
            pl.BlockSpec((ci, _RB, 128), lambda i: (0, i, 0)),
            pl.BlockSpec((ci, _RB, 128), lambda i: (0, i, 0)),
            pl.BlockSpec((_NC, _RB, 128), lambda i: (0, i, 0)),
            pl.BlockSpec((IN_CH, HID), lambda i: (0, 0)),
            pl.BlockSpec((1, HID), lambda i: (0, 0)),
            pl.BlockSpec((1, HID), lambda i: (0, 0)),
            pl.BlockSpec((1, HID), lambda i: (0, 0)),
            pl.BlockSpec((HID, OUT_CH), lambda i: (0, 0)),
        ],
        out_specs=pl.BlockSpec((c2, _RB, 128), lambda i: (0, i, 0)),
        out_shape=jax.ShapeDtypeStruct((c2, N_PAD, 128), jnp.float32),
    )(sxc, xsc, deg16, W1, b1.reshape(1, HID), g1.reshape(1, HID),
      be1.reshape(1, HID), W2)


def _tc_c_body(agg_ref, hs_ref, deg_ref, b_ref, g_ref, be_ref, o_ref):
    dinv = lax.rsqrt(deg_ref[0, :, 0:1] + deg_ref[1, :, 0:1] + 1.0)
    t = (agg_ref[...] + hs_ref[...]) * dinv[None] + b_ref[...][:, None, :]
    mu = jnp.mean(t, axis=(0, 2))[None, :, None]
    var = jnp.mean((t - mu) ** 2, axis=(0, 2))[None, :, None]
    tn = (t - mu) * lax.rsqrt(var + EPS)
    tn = jnp.maximum(tn * g_ref[...][:, None, :] + be_ref[...][:, None, :], 0.0)
    o_ref[...] = jnp.concatenate([tn[0], tn[1]], axis=1)


def _tc_c(agg2c, hs2c, deg16, b2, g2, be2):
    c2 = OUT_CH // 128
    return pl.pallas_call(
        _tc_c_body,
        grid=(_RG,),
        in_specs=[
            pl.BlockSpec((c2, _RB, 128), lambda i: (0, i, 0)),
            pl.BlockSpec((c2, _RB, 128), lambda i: (0, i, 0)),
            pl.BlockSpec((_NC, _RB, 128), lambda i: (0, i, 0)),
            pl.BlockSpec((c2, 128), lambda i: (0, 0)),
            pl.BlockSpec((c2, 128), lambda i: (0, 0)),
            pl.BlockSpec((c2, 128), lambda i: (0, 0)),
        ],
        out_specs=pl.BlockSpec((_RB, OUT_CH), lambda i: (i, 0)),
        out_shape=jax.ShapeDtypeStruct((N_PAD, OUT_CH), jnp.float32),
    )(agg2c, hs2c, deg16, b2.reshape(c2, 128), g2.reshape(c2, 128),
      be2.reshape(c2, 128))


def kernel(x, edge_index, W1, b1, g1, be1, W2, b2, g2, be2):
    ei = edge_index.astype(jnp.int32)
    pad = jnp.full((E_PAD - E,), N, jnp.int32)
    src2d = jnp.pad(jnp.concatenate([ei[0], pad]).reshape(E_PAD // _EB, _EB),
                    ((0, 2), (0, 0)))
    dst2d = jnp.concatenate([ei[1], pad]).reshape(E_PAD // _EB, _EB)
    x_p = jnp.pad(x, ((0, N_PAD - N), (0, 0)))
    zeros128 = jnp.zeros((_ROWS_PER_SUB, 128), jnp.float32)
    ones128 = jnp.ones((_EB, 128), jnp.float32)

    deg16 = _sc_degree(dst2d, zeros128, ones128)
    xsc = _tc_a(x_p, deg16)
    sxc = _sc_segsum(xsc, src2d, dst2d, zeros128, IN_CH // 128)
    hs2c = _tc_b(sxc, xsc, deg16, W1, b1, g1, be1, W2)
    agg2c = _sc_segsum(hs2c, src2d, dst2d, zeros128, OUT_CH // 128)
    out_p = _tc_c(agg2c, hs2c, deg16, b2, g2, be2)
    return out_p[:N]


# speedup vs baseline: 1.0376x; 1.0015x over previous
import functools
import jax
import jax.numpy as jnp
from jax import lax
from jax.experimental import pallas as pl
from jax.experimental.pallas import tpu as pltpu
from jax.experimental.pallas import tpu_sc as plsc

N = 10000
N_PAD = 10240
E = 160000
E_PAD = 163840
IN_CH = 256
HID = 512
OUT_CH = 256
EPS = 1e-5

_NC = 2
_NS = 16
_ROWS_PER_SUB = N_PAD // _NS
_EDGES_PER_SUB = E_PAD // _NS
_EB = 128
_NB = _EDGES_PER_SUB // _EB


def _sc_mesh():
    return plsc.VectorSubcoreMesh(core_axis_name="c", subcore_axis_name="s")


def _sc_degree(dst2d, zeros128, ones128):
    half = _NB // 2

    @functools.partial(
        pl.kernel,
        out_type=jax.ShapeDtypeStruct((_NC, N_PAD, 128), jnp.float32),
        mesh=_sc_mesh(),
        scratch_types=[
            pltpu.VMEM_SHARED((N_PAD, 128), jnp.float32),
            pltpu.VMEM((_EB, 128), jnp.float32),
            pltpu.VMEM((half, _EB), jnp.int32),
            pltpu.SemaphoreType.DMA,
        ],
    )
    def k(dst_hbm, z_hbm, ones_hbm, out_hbm, acc, ones_v, didx2, ssem):
        cid = lax.axis_index("c")
        sid = lax.axis_index("s")
        pltpu.sync_copy(ones_hbm, ones_v)
        pltpu.sync_copy(dst_hbm.at[pl.ds((cid * _NS + sid) * half, half)],
                        didx2)
        pltpu.sync_copy(z_hbm, acc.at[pl.ds(sid * _ROWS_PER_SUB, _ROWS_PER_SUB)])
        plsc.subcore_barrier()

        @pl.loop(0, half, step=8)
        def _(b):
            cps = [pltpu.async_copy(ones_v, acc.at[didx2.at[b + j]], ssem,
                                    add=True) for j in range(8)]
            for cp in cps:
                cp.wait()

        plsc.subcore_barrier()
        s = pl.ds(sid * _ROWS_PER_SUB, _ROWS_PER_SUB)
        pltpu.sync_copy(acc.at[s], out_hbm.at[cid].at[s])

    return k(dst2d, zeros128, ones128)


def _split_gather(table, sidx, bufs, j, sem):
    pltpu.async_copy(table.at[sidx.at[j]], bufs.at[j], sem)


def _sc_segsum(hs_c, src2d, dst2d, zeros128, n_chunks):
    per_core = n_chunks // _NC

    @functools.partial(
        pl.kernel,
        out_type=jax.ShapeDtypeStruct((n_chunks, N_PAD, 128), jnp.float32),
        mesh=_sc_mesh(),
        scratch_types=[
            pltpu.VMEM_SHARED((N_PAD, 128), jnp.float32),
            pltpu.VMEM((2, _EB, 128), jnp.float32),
            pltpu.VMEM((_NB, _EB), jnp.int32),
            pltpu.VMEM((2, _EB), jnp.int32),
            [pltpu.SemaphoreType.DMA] * 2,
            [pltpu.SemaphoreType.DMA] * 2,
            [pltpu.SemaphoreType.DMA] * 2,
        ],
    )
    def k(hs_hbm, src_hbm, dst_hbm, z_hbm, out_hbm, acc, bufs, didx2, sidx,
          gsems, ssems, isems):
        cid = lax.axis_index("c")
        sid = lax.axis_index("s")
        rows = pl.ds(sid * _ROWS_PER_SUB, _ROWS_PER_SUB)
        ib = sid * _NB
        pltpu.sync_copy(dst_hbm.at[pl.ds(ib, _NB)], didx2)
        for ci in range(per_core):
            chunk = cid * per_core + ci
            pltpu.sync_copy(z_hbm, acc.at[rows])
            for j in range(2):
                pltpu.sync_copy(src_hbm.at[ib + j], sidx.at[j])
                _split_gather(hs_hbm.at[chunk], sidx, bufs, j, gsems[j])
            plsc.subcore_barrier()

            @pl.loop(0, _NB, step=2)
            def _(b):
                scs, lds = [], []
                for j in range(2):
                    pltpu.make_async_copy(hs_hbm.at[chunk].at[sidx.at[j]],
                                          bufs.at[j], gsems[j]).wait()
                    lds.append(pltpu.async_copy(src_hbm.at[ib + b + 2 + j],
                                                sidx.at[j], isems[j]))
                    scs.append(pltpu.async_copy(
                        bufs.at[j], acc.at[didx2.at[b + j]], ssems[j],
                        add=True))
                for j in range(2):
                    scs[j].wait()
                    lds[j].wait()

                    @pl.when(b + 2 + j < _NB)
                    def _():
                        _split_gather(hs_hbm.at[chunk], sidx, bufs, j,
                                      gsems[j])

            plsc.subcore_barrier()
            pltpu.sync_copy(acc.at[rows], out_hbm.at[chunk].at[rows])

    return k(hs_c, src2d, dst2d, zeros128)


_RB = 1280
_RG = N_PAD // _RB


def _tc_a_body(x_ref, deg_ref, o_ref):
    dinv = lax.rsqrt(deg_ref[0, :, 0:1] + deg_ref[1, :, 0:1] + 1.0)
    o_ref[0] = x_ref[...] * dinv


def _tc_a(x_p, deg16):
    return pl.pallas_call(
        _tc_a_body,
        grid=(_RG, IN_CH // 128),
        in_specs=[
            pl.BlockSpec((_RB, 128), lambda i, j: (i, j)),
            pl.BlockSpec((_NC, _RB, 128), lambda i, j: (0, i, 0)),
        ],
        out_specs=pl.BlockSpec((1, _RB, 128), lambda i, j: (j, i, 0)),
        out_shape=jax.ShapeDtypeStruct((IN_CH // 128, N_PAD, 128), jnp.float32),
    )(x_p, deg16)


def _tc_b_body(sx_ref, xs_ref, deg_ref, w1_ref, b_ref, g_ref, be_ref, w2_ref,
               o_ref):
    dinv = lax.rsqrt(deg_ref[0, :, 0:1] + deg_ref[1, :, 0:1] + 1.0)
    u = (sx_ref[...] + xs_ref[...]) * dinv[None]
    t = jnp.dot(u[0], w1_ref[:128], preferred_element_type=jnp.float32)
    t = t + jnp.dot(u[1], w1_ref[128:], preferred_element_type=jnp.float32)
    t = t + b_ref[...]
    mu = jnp.mean(t, axis=-1, keepdims=True)
    var = jnp.mean((t - mu) ** 2, axis=-1, keepdims=True)
    tn = (t - mu) * lax.rsqrt(var + EPS) * g_ref[...] + be_ref[...]
    tn = jnp.maximum(tn, 0.0)
    h2 = jnp.dot(tn, w2_ref[...], preferred_element_type=jnp.float32)
    row = lax.broadcasted_iota(jnp.int32, (_RB, 1), 0) + pl.program_id(0) * _RB
    hs2 = h2 * dinv * (row < N).astype(jnp.float32)
    o_ref[0] = hs2[:, :128]
    o_ref[1] = hs2[:, 128:]


def _tc_b(sxc, xsc, deg16, W1, b1, g1, be1, W2):
    ci = IN_CH // 128
    c2 = OUT_CH // 128
    return pl.pallas_call(
        _tc_b_body,
        grid=(_RG,),
        in_specs=[
            pl.BlockSpec((ci, _RB, 128), lambda i: (0, i, 0)),
            pl.BlockSpec((ci, _RB, 128), lambda i: (0, i, 0)),
            pl.BlockSpec((_NC, _RB, 128), lambda i: (0, i, 0)),
            pl.BlockSpec((IN_CH, HID), lambda i: (0, 0)),
            pl.BlockSpec((1, HID), lambda i: (0, 0)),
            pl.BlockSpec((1, HID), lambda i: (0, 0)),
            pl.BlockSpec((1, HID), lambda i: (0, 0)),
            pl.BlockSpec((HID, OUT_CH), lambda i: (0, 0)),
        ],
        out_specs=pl.BlockSpec((c2, _RB, 128), lambda i: (0, i, 0)),
        out_shape=jax.ShapeDtypeStruct((c2, N_PAD, 128), jnp.float32),
    )(sxc, xsc, deg16, W1, b1.reshape(1, HID), g1.reshape(1, HID),
      be1.reshape(1, HID), W2)


def _tc_c_body(agg_ref, hs_ref, deg_ref, b_ref, g_ref, be_ref, o_ref):
    dinv = lax.rsqrt(deg_ref[0, :, 0:1] + deg_ref[1, :, 0:1] + 1.0)
    t = (agg_ref[...] + hs_ref[...]) * dinv[None] + b_ref[...][:, None, :]
    mu = jnp.mean(t, axis=(0, 2))[None, :, None]
    var = jnp.mean((t - mu) ** 2, axis=(0, 2))[None, :, None]
    tn = (t - mu) * lax.rsqrt(var + EPS)
    tn = jnp.maximum(tn * g_ref[...][:, None, :] + be_ref[...][:, None, :], 0.0)
    o_ref[...] = jnp.concatenate([tn[0], tn[1]], axis=1)


def _tc_c(agg2c, hs2c, deg16, b2, g2, be2):
    c2 = OUT_CH // 128
    return pl.pallas_call(
        _tc_c_body,
        grid=(_RG,),
        in_specs=[
            pl.BlockSpec((c2, _RB, 128), lambda i: (0, i, 0)),
            pl.BlockSpec((c2, _RB, 128), lambda i: (0, i, 0)),
            pl.BlockSpec((_NC, _RB, 128), lambda i: (0, i, 0)),
            pl.BlockSpec((c2, 128), lambda i: (0, 0)),
            pl.BlockSpec((c2, 128), lambda i: (0, 0)),
            pl.BlockSpec((c2, 128), lambda i: (0, 0)),
        ],
        out_specs=pl.BlockSpec((_RB, OUT_CH), lambda i: (i, 0)),
        out_shape=jax.ShapeDtypeStruct((N_PAD, OUT_CH), jnp.float32),
    )(agg2c, hs2c, deg16, b2.reshape(c2, 128), g2.reshape(c2, 128),
      be2.reshape(c2, 128))


def kernel(x, edge_index, W1, b1, g1, be1, W2, b2, g2, be2):
    ei = edge_index.astype(jnp.int32)
    pad = jnp.full((E_PAD - E,), N, jnp.int32)
    src2d = jnp.pad(jnp.concatenate([ei[0], pad]).reshape(E_PAD // _EB, _EB),
                    ((0, 2), (0, 0)))
    dst2d = jnp.concatenate([ei[1], pad]).reshape(E_PAD // _EB, _EB)
    x_p = jnp.pad(x, ((0, N_PAD - N), (0, 0)))
    zeros128 = jnp.zeros((_ROWS_PER_SUB, 128), jnp.float32)
    ones128 = jnp.ones((_EB, 128), jnp.float32)

    deg16 = _sc_degree(dst2d, zeros128, ones128)
    xsc = _tc_a(x_p, deg16)
    sxc = _sc_segsum(xsc, src2d, dst2d, zeros128, IN_CH // 128)
    hs2c = _tc_b(sxc, xsc, deg16, W1, b1, g1, be1, W2)
    agg2c = _sc_segsum(hs2c, src2d, dst2d, zeros128, OUT_CH // 128)
    out_p = _tc_c(agg2c, hs2c, deg16, b2, g2, be2)
    return out_p[:N]
